# baseline (speedup 1.0000x reference)
import functools

import jax
import jax.numpy as jnp
from jax import lax
from jax.experimental import pallas as pl
from jax.experimental.pallas import tpu as pltpu
from jax.experimental.pallas import tpu_sc as plsc

_CONTEXT_WINDOW = 4096


def _emb_gather_sc(emb_hbm, tok_hbm, out_hbm, idx_v, row_v, sem):
    cid = lax.axis_index("c")
    sid = lax.axis_index("s")

    @pl.when((cid == 0) & (sid == 0))
    def _():
        pltpu.sync_copy(tok_hbm, idx_v)
        pltpu.async_copy(emb_hbm.at[idx_v], row_v, sem).wait()
        pltpu.sync_copy(row_v, out_hbm)


def _emb_row_sc(emb, token, d):
    tok8 = jnp.full((8,), token, jnp.int32)
    mesh = plsc.VectorSubcoreMesh(core_axis_name="c", subcore_axis_name="s")
    return pl.kernel(
        _emb_gather_sc,
        out_type=jax.ShapeDtypeStruct((8, d), emb.dtype),
        mesh=mesh,
        scratch_types=[
            pltpu.VMEM((8,), jnp.int32),
            pltpu.VMEM((8, d), emb.dtype),
            pltpu.SemaphoreType.DMA,
        ],
    )(emb, tok8)


def _shift_copy_kernel(idea_ref, row_ref, out_ref, carry_ref):
    j = pl.program_id(1)
    r = idea_ref.shape[1]
    out_ref[0, 0:r - 1, :] = idea_ref[0, 1:r, :]

    @pl.when(j == 0)
    def _():
        out_ref[0, r - 1:r, :] = row_ref[0:1, :]

    @pl.when(j != 0)
    def _():
        out_ref[0, r - 1:r, :] = carry_ref[...]

    carry_ref[...] = idea_ref[0, 0:1, :]


def kernel(idea, token, emb):
    b, l, d = idea.shape
    lout = min(_CONTEXT_WINDOW, l + 1)
    if lout == l + 1:
        idea = jnp.concatenate([jnp.zeros((b, 1, d), idea.dtype), idea], axis=1)
        l = lout
    r = 1024 if l % 1024 == 0 else l
    nb = l // r
    emb_row = _emb_row_sc(emb, token, d)
    out = pl.pallas_call(
        _shift_copy_kernel,
        grid=(b, nb),
        in_specs=[
            pl.BlockSpec((1, r, d), lambda bb, j: (bb, nb - 1 - j, 0)),
            pl.BlockSpec((8, d), lambda bb, j: (0, 0)),
        ],
        out_specs=pl.BlockSpec((1, r, d), lambda bb, j: (bb, nb - 1 - j, 0)),
        scratch_shapes=[pltpu.VMEM((1, d), idea.dtype)],
        out_shape=jax.ShapeDtypeStruct((b, l, d), idea.dtype),
        compiler_params=pltpu.CompilerParams(
            dimension_semantics=("parallel", "arbitrary"),
            vmem_limit_bytes=100 * 1024 * 1024,
        ),
    )(idea, emb_row)
    return out
